# baseline (speedup 1.0000x reference)
import functools

import jax
import jax.numpy as jnp
from jax import lax
from jax.experimental import pallas as pl
from jax.experimental.pallas import tpu as pltpu
from jax.experimental.pallas import tpu_sc as plsc

D = 64
L = 16
NC, NS = 2, 16
NW = NC * NS
SUB = 128
NSUB = 4
CHUNK = SUB * NSUB
SCALE = 8.0


@functools.lru_cache(maxsize=None)
def _make(n_total):
    per_w = n_total // NW
    nchunk = per_w // CHUNK
    mesh = plsc.VectorSubcoreMesh(core_axis_name="c", subcore_axis_name="s")

    @functools.partial(
        pl.kernel,
        mesh=mesh,
        out_type=jax.ShapeDtypeStruct((n_total, D), jnp.float32),
        scratch_types=[
            pltpu.VMEM((NSUB, SUB), jnp.int32),
            pltpu.VMEM((CHUNK, D), jnp.float32),
            pltpu.SemaphoreType.DMA,
        ],
        compiler_params=pltpu.CompilerParams(use_tc_tiling_on_sc=False),
    )
    def k(x_hbm, tab_hbm, out_hbm, idx_v, rows_v, sem):
        wid = lax.axis_index("s") * NC + lax.axis_index("c")
        base_row = wid * (per_w // SUB)
        base_out = wid * per_w

        def step(i, carry):
            pltpu.sync_copy(x_hbm.at[pl.ds(base_row + i * NSUB, NSUB)], idx_v)
            cps = [
                pltpu.async_copy(
                    tab_hbm.at[idx_v.at[j]],
                    rows_v.at[pl.ds(j * SUB, SUB)],
                    sem,
                )
                for j in range(NSUB)
            ]
            for cp in cps:
                cp.wait()

            def srow(r, c2):
                for q in range(16):
                    rr = r * 4 + q // 4
                    cc = (q % 4) * L
                    rows_v[rr, pl.ds(cc, L)] = rows_v[rr, pl.ds(cc, L)] * SCALE
                return c2

            lax.fori_loop(0, CHUNK // 4, srow, 0)
            pltpu.sync_copy(rows_v, out_hbm.at[pl.ds(base_out + i * CHUNK, CHUNK)])
            return carry

        lax.fori_loop(0, nchunk, step, 0)

    return k


@jax.jit
def kernel(x, input_embedding):
    b, h = x.shape
    n = b * h
    xf = x.reshape(n // SUB, SUB).astype(jnp.int32)
    out = _make(n)(xf, input_embedding)
    return out.reshape(b, h, D)


# speedup vs baseline: 1.0889x; 1.0889x over previous
import functools

import jax
import jax.numpy as jnp
from jax import lax
from jax.experimental import pallas as pl
from jax.experimental.pallas import tpu as pltpu
from jax.experimental.pallas import tpu_sc as plsc

D = 64
L = 16
NC, NS = 2, 16
NW = NC * NS
SUB = 128
NSUB = 2
CHUNK = SUB * NSUB
NBUF = 4
SCALE = 8.0


@functools.lru_cache(maxsize=None)
def _make(n_total):
    per_w = n_total // NW
    nrow_w = per_w // SUB
    nchunk = per_w // CHUNK
    assert nchunk % NBUF == 0
    mesh = plsc.VectorSubcoreMesh(core_axis_name="c", subcore_axis_name="s")

    @functools.partial(
        pl.kernel,
        mesh=mesh,
        out_type=jax.ShapeDtypeStruct((n_total, D), jnp.float32),
        scratch_types=[
            pltpu.VMEM((nrow_w, SUB), jnp.int32),
            pltpu.VMEM((NBUF, CHUNK, D), jnp.float32),
        ]
        + [pltpu.SemaphoreType.DMA] * (2 * NBUF),
        compiler_params=pltpu.CompilerParams(use_tc_tiling_on_sc=False),
    )
    def k(x_hbm, tab_hbm, out_hbm, idx_all, rows, *sems):
        gsem = sems[:NBUF]
        osem = sems[NBUF:]
        wid = lax.axis_index("s") * NC + lax.axis_index("c")
        base_row = wid * nrow_w
        base_out = wid * per_w

        pltpu.sync_copy(x_hbm.at[pl.ds(base_row, nrow_w)], idx_all)

        def fire_gather(i, b):
            for j in range(NSUB):
                pltpu.async_copy(
                    tab_hbm.at[idx_all.at[i * NSUB + j]],
                    rows.at[b, pl.ds(j * SUB, SUB)],
                    gsem[b],
                )

        def wait_gather(i, b):
            for j in range(NSUB):
                pltpu.make_async_copy(
                    tab_hbm.at[idx_all.at[i * NSUB + j]],
                    rows.at[b, pl.ds(j * SUB, SUB)],
                    gsem[b],
                ).wait()

        def fire_out(i, b):
            pltpu.async_copy(
                rows.at[b], out_hbm.at[pl.ds(base_out + i * CHUNK, CHUNK)], osem[b]
            )

        def wait_out(i, b):
            pltpu.make_async_copy(
                rows.at[b], out_hbm.at[pl.ds(base_out + i * CHUNK, CHUNK)], osem[b]
            ).wait()

        def scale(b):
            def srow(r, c):
                for q in range(16):
                    rr = r * 4 + q // 4
                    cc = (q % 4) * L
                    rows[b, rr, pl.ds(cc, L)] = rows[b, rr, pl.ds(cc, L)] * SCALE
                return c

            lax.fori_loop(0, CHUNK // 4, srow, 0)

        for b in range(NBUF - 1):
            fire_gather(b, b)

        def group(g, c):
            for b in range(NBUF):
                i = g * NBUF + b
                j = i + NBUF - 1
                bj = (b + NBUF - 1) % NBUF
                if b == 0:
                    @pl.when(g >= 1)
                    def _():
                        wait_out(i - 1, bj)

                    fire_gather(j, bj)
                else:
                    @pl.when(j < nchunk)
                    def _():
                        wait_out(i - 1, bj)
                        fire_gather(j, bj)

                wait_gather(i, b)
                scale(b)
                fire_out(i, b)
            return c

        lax.fori_loop(0, nchunk // NBUF, group, 0)

        for b in range(NBUF):
            i = nchunk - NBUF + b
            wait_out(i, i % NBUF)

    return k


@jax.jit
def kernel(x, input_embedding):
    b, h = x.shape
    n = b * h
    xf = x.reshape(n // SUB, SUB).astype(jnp.int32)
    out = _make(n)(xf, input_embedding)
    return out.reshape(b, h, D)
